# Initial kernel scaffold; baseline (speedup 1.0000x reference)
import jax
import jax.numpy as jnp
from jax.experimental import pallas as pl


def kernel(z, d_ij, idx_i, idx_j, emb, Win, Wf1, bf1, Wf2, bf2, Wo1, bo1, Wo2, bo2):
    raise NotImplementedError("write your pallas kernel here")


# speedup vs baseline: 1.6960x; 1.6960x over previous
import functools
import math

import jax
import jax.numpy as jnp
from jax import lax
from jax.experimental import pallas as pl
from jax.experimental.pallas import tpu as pltpu
from jax.experimental.pallas import tpu_sc as plsc

_N = 50000
_E = 800000
_D = 64
_NRBF = 50
_CUTOFF = 5.0
_ZMAX = 100
_L = 3
_H = 32
_LN2 = math.log(2.0)

_NPAD = 51200
_EPAD = 811008
_NCORES = 2
_NSUB = 16
_CH = 96
_NBUF = 4
_ET = _EPAD // _NSUB
_NC = _ET // _CH
_NG = _NC // _NBUF
_NROWS_T = _NPAD // _NSUB

_BN = 2048
_BE = 4096

_RBF_WIDTH = _CUTOFF / (_NRBF - 1)
_RBF_COEFF = -0.5 / (_RBF_WIDTH * _RBF_WIDTH)


def _ssp(x):
    return jnp.maximum(x, 0.0) + jnp.log(1.0 + jnp.exp(-jnp.abs(x))) - _LN2


def _filter_body(d_ref, off_ref, wf1_ref, bf1_ref, wf2_ref, bf2_ref, *out_refs):
    d = d_ref[...]
    off = off_ref[...]
    f = jnp.exp(_RBF_COEFF * (d - off) ** 2)
    rc = 0.5 * (jnp.cos(d * (math.pi / _CUTOFF)) + 1.0)
    rc = rc * (d < _CUTOFF).astype(jnp.float32)
    for l in range(_L):
        h = jnp.dot(f, wf1_ref[l], preferred_element_type=jnp.float32)
        h = _ssp(h + bf1_ref[l])
        w = jnp.dot(h, wf2_ref[l], preferred_element_type=jnp.float32)
        w = (w + bf2_ref[l]) * rc
        out_refs[2 * l][...] = w[:, :_H]
        out_refs[2 * l + 1][...] = w[:, _H:]


def _filters(d2, offs, wf1p, bf1, wf2, bf2):
    nblk = _EPAD // _BE
    full = lambda shape: pl.BlockSpec(shape, lambda i: (0,) * len(shape))
    return pl.pallas_call(
        _filter_body,
        grid=(nblk,),
        in_specs=[
            pl.BlockSpec((_BE, 1), lambda i: (i, 0)),
            full((1, 64)),
            full((_L, 64, _D)),
            full((_L, _D)),
            full((_L, _D, _D)),
            full((_L, _D)),
        ],
        out_specs=[pl.BlockSpec((_BE, _H), lambda i: (i, 0))] * (2 * _L),
        out_shape=[jax.ShapeDtypeStruct((_EPAD, _H), jnp.float32)] * (2 * _L),
    )(d2, offs, wf1p, bf1, wf2, bf2)


def _embed_body(z_ref, emb_ref, win_ref, x_ref, v0_ref, v1_ref):
    z = z_ref[...]
    ids = lax.broadcasted_iota(jnp.int32, (_BN, 128), 1)
    oh = (z == ids).astype(jnp.float32)
    x = jnp.dot(oh, emb_ref[...], preferred_element_type=jnp.float32)
    v = jnp.dot(x, win_ref[...], preferred_element_type=jnp.float32)
    x_ref[...] = x
    v0_ref[...] = v[:, :_H]
    v1_ref[...] = v[:, _H:]


def _embed(z2, emb_pad, win0):
    nblk = _NPAD // _BN
    full = lambda shape: pl.BlockSpec(shape, lambda i: (0,) * len(shape))
    return pl.pallas_call(
        _embed_body,
        grid=(nblk,),
        in_specs=[
            pl.BlockSpec((_BN, 1), lambda i: (i, 0)),
            full((128, _D)),
            full((_D, _D)),
        ],
        out_specs=[
            pl.BlockSpec((_BN, _D), lambda i: (i, 0)),
            pl.BlockSpec((_BN, _H), lambda i: (i, 0)),
            pl.BlockSpec((_BN, _H), lambda i: (i, 0)),
        ],
        out_shape=[
            jax.ShapeDtypeStruct((_NPAD, _D), jnp.float32),
            jax.ShapeDtypeStruct((_NPAD, _H), jnp.float32),
            jax.ShapeDtypeStruct((_NPAD, _H), jnp.float32),
        ],
    )(z2, emb_pad, win0)


def _node_body(has_next, x_ref, a0_ref, a1_ref, wo1_ref, bo1_ref, wo2_ref,
               bo2_ref, *rest):
    if has_next:
        win_ref, xn_ref, v0_ref, v1_ref = rest
    else:
        (xn_ref,) = rest
    agg = jnp.concatenate([a0_ref[...], a1_ref[...]], axis=1)
    h = _ssp(jnp.dot(agg, wo1_ref[...], preferred_element_type=jnp.float32)
             + bo1_ref[...])
    vout = jnp.dot(h, wo2_ref[...], preferred_element_type=jnp.float32)
    xn = x_ref[...] + vout + bo2_ref[...]
    xn_ref[...] = xn
    if has_next:
        v = jnp.dot(xn, win_ref[...], preferred_element_type=jnp.float32)
        v0_ref[...] = v[:, :_H]
        v1_ref[...] = v[:, _H:]


def _node(x, a0, a1, wo1, bo1, wo2, bo2, win_next=None):
    nblk = _NPAD // _BN
    full = lambda shape: pl.BlockSpec(shape, lambda i: (0,) * len(shape))
    has_next = win_next is not None
    in_specs = [
        pl.BlockSpec((_BN, _D), lambda i: (i, 0)),
        pl.BlockSpec((_BN, _H), lambda i: (i, 0)),
        pl.BlockSpec((_BN, _H), lambda i: (i, 0)),
        full((_D, _D)),
        full((1, _D)),
        full((_D, _D)),
        full((1, _D)),
    ]
    out_specs = [pl.BlockSpec((_BN, _D), lambda i: (i, 0))]
    out_shape = [jax.ShapeDtypeStruct((_NPAD, _D), jnp.float32)]
    args = [x, a0, a1, wo1, bo1, wo2, bo2]
    if has_next:
        in_specs.append(full((_D, _D)))
        out_specs += [pl.BlockSpec((_BN, _H), lambda i: (i, 0))] * 2
        out_shape += [jax.ShapeDtypeStruct((_NPAD, _H), jnp.float32)] * 2
        args.append(win_next)
    return pl.pallas_call(
        functools.partial(_node_body, has_next),
        grid=(nblk,),
        in_specs=in_specs,
        out_specs=out_specs,
        out_shape=out_shape,
    )(*args)


def _conv_body(v0, v1, w0, w1, idxi_hbm, idxj_hbm, zeros_hbm, a0, a1,
               idxiv, idxjv, xjv, wv, acc, semI, semG, semW, semS):
    cid = lax.axis_index("c")
    sid = lax.axis_index("s")
    tbase = sid * _ET
    r0 = sid * _NROWS_T

    def run(vtab, wtab, out):
        pltpu.sync_copy(zeros_hbm.at[pl.ds(r0, _NROWS_T)],
                        acc.at[pl.ds(r0, _NROWS_T)])
        plsc.subcore_barrier()

        def start_idx(j, b):
            pltpu.async_copy(idxi_hbm.at[pl.ds(tbase + j * _CH, _CH)],
                             idxiv.at[b], semI.at[b])
            pltpu.async_copy(idxj_hbm.at[pl.ds(tbase + j * _CH, _CH)],
                             idxjv.at[b], semI.at[b])

        def wait_idx(j, b):
            pltpu.make_async_copy(idxi_hbm.at[pl.ds(tbase + j * _CH, _CH)],
                                  idxiv.at[b], semI.at[b]).wait()
            pltpu.make_async_copy(idxj_hbm.at[pl.ds(tbase + j * _CH, _CH)],
                                  idxjv.at[b], semI.at[b]).wait()

        def start_gw(j, b):
            pltpu.async_copy(vtab.at[idxjv.at[b]], xjv.at[b], semG.at[b])
            pltpu.async_copy(wtab.at[pl.ds(tbase + j * _CH, _CH)],
                             wv.at[b], semW.at[b])

        def wait_gw(j, b):
            pltpu.make_async_copy(vtab.at[idxjv.at[b]], xjv.at[b],
                                  semG.at[b]).wait()
            pltpu.make_async_copy(wtab.at[pl.ds(tbase + j * _CH, _CH)],
                                  wv.at[b], semW.at[b]).wait()

        def compute(b):
            @pl.loop(0, _CH, unroll=8)
            def _mul(i):
                for h0 in (0, 16):
                    xjv[b, i, pl.ds(h0, 16)] = (
                        xjv[b, i, pl.ds(h0, 16)] * wv[b, i, pl.ds(h0, 16)])

        def start_scat(b):
            pltpu.async_copy(xjv.at[b], acc.at[idxiv.at[b]], semS.at[b],
                             add=True)

        def wait_scat(b):
            pltpu.make_async_copy(xjv.at[b], acc.at[idxiv.at[b]],
                                  semS.at[b]).wait()

        def steady(j, b):
            b1, b2 = (b + 1) % _NBUF, (b + 2) % _NBUF
            wait_gw(j, b)
            compute(b)
            start_scat(b)
            wait_scat(b2)
            start_idx(j + 2, b2)
            wait_idx(j + 1, b1)
            start_gw(j + 1, b1)

        start_idx(0, 0)
        start_idx(1, 1)
        wait_idx(0, 0)
        start_gw(0, 0)
        for j in (0, 1):
            wait_gw(j, j)
            compute(j)
            start_scat(j)
            start_idx(j + 2, j + 2)
            wait_idx(j + 1, j + 1)
            start_gw(j + 1, j + 1)
        steady(2, 2)
        steady(3, 3)

        @pl.loop(1, _NG - 1)
        def _grp(g):
            j0 = g * _NBUF
            for b in range(_NBUF):
                steady(j0 + b, b)

        steady(_NC - 4, 0)
        steady(_NC - 3, 1)
        wait_gw(_NC - 2, 2)
        compute(2)
        start_scat(2)
        wait_scat(0)
        wait_idx(_NC - 1, 3)
        start_gw(_NC - 1, 3)
        wait_gw(_NC - 1, 3)
        compute(3)
        start_scat(3)
        wait_scat(1)
        wait_scat(2)
        wait_scat(3)

        plsc.subcore_barrier()
        pltpu.sync_copy(acc.at[pl.ds(r0, _NROWS_T)],
                        out.at[pl.ds(r0, _NROWS_T)])

    @pl.when(cid == 0)
    def _():
        run(v0, w0, a0)

    @pl.when(cid == 1)
    def _():
        run(v1, w1, a1)


def _conv(v0, v1, w0, w1, idxi, idxj, zeros_h):
    mesh = plsc.VectorSubcoreMesh(core_axis_name="c", subcore_axis_name="s",
                                  num_cores=_NCORES, num_subcores=_NSUB)
    f = pl.kernel(
        _conv_body,
        out_type=[jax.ShapeDtypeStruct((_NPAD, _H), jnp.float32)] * 2,
        mesh=mesh,
        compiler_params=pltpu.CompilerParams(use_tc_tiling_on_sc=False),
        scratch_types=[
            pltpu.VMEM((_NBUF, _CH), jnp.int32),
            pltpu.VMEM((_NBUF, _CH), jnp.int32),
            pltpu.VMEM((_NBUF, _CH, _H), jnp.float32),
            pltpu.VMEM((_NBUF, _CH, _H), jnp.float32),
            pltpu.VMEM_SHARED((_NPAD, _H), jnp.float32),
            pltpu.SemaphoreType.DMA((_NBUF,)),
            pltpu.SemaphoreType.DMA((_NBUF,)),
            pltpu.SemaphoreType.DMA((_NBUF,)),
            pltpu.SemaphoreType.DMA((_NBUF,)),
        ],
    )
    return f(v0, v1, w0, w1, idxi, idxj, zeros_h)


def kernel(z, d_ij, idx_i, idx_j, emb, Win, Wf1, bf1, Wf2, bf2, Wo1, bo1,
           Wo2, bo2):
    f32, i32 = jnp.float32, jnp.int32
    d2 = jnp.concatenate(
        [d_ij.astype(f32), jnp.full((_EPAD - _E,), 2.0 * _CUTOFF, f32)]
    ).reshape(_EPAD, 1)
    idxi = jnp.concatenate([idx_i.astype(i32),
                            jnp.zeros((_EPAD - _E,), i32)])
    idxj = jnp.concatenate([idx_j.astype(i32),
                            jnp.zeros((_EPAD - _E,), i32)])
    z2 = jnp.concatenate([z.astype(i32),
                          jnp.zeros((_NPAD - _N,), i32)]).reshape(_NPAD, 1)
    offs = jnp.concatenate([
        jnp.linspace(0.0, _CUTOFF, _NRBF, dtype=f32),
        jnp.full((64 - _NRBF,), 1e3, f32),
    ]).reshape(1, 64)
    wf1p = jnp.pad(Wf1.astype(f32), ((0, 0), (0, 64 - _NRBF), (0, 0)))
    emb_pad = jnp.pad(emb.astype(f32), ((0, 128 - _ZMAX), (0, 0)))
    zeros_h = jnp.zeros((_NPAD, _H), f32)

    wh = _filters(d2, offs, wf1p, bf1.astype(f32), Wf2.astype(f32),
                  bf2.astype(f32))
    x, v0, v1 = _embed(z2, emb_pad, Win[0].astype(f32))
    for l in range(_L):
        a0, a1 = _conv(v0, v1, wh[2 * l], wh[2 * l + 1], idxi, idxj, zeros_h)
        win_next = Win[l + 1].astype(f32) if l + 1 < _L else None
        outs = _node(x, a0, a1, Wo1[l].astype(f32), bo1[l].reshape(1, _D),
                     Wo2[l].astype(f32), bo2[l].reshape(1, _D), win_next)
        if win_next is not None:
            x, v0, v1 = outs
        else:
            (x,) = outs
    return x[:_N]
